# baseline (speedup 1.0000x reference)
import functools

import jax
import jax.numpy as jnp
from jax import lax
from jax.experimental import pallas as pl
from jax.experimental.pallas import tpu as pltpu
from jax.experimental.pallas import tpu_sc as plsc

N_OBJ = 2048
NUM_CLS = 151
HIDDEN = 256
P = 131072
NN = N_OBJ * N_OBJ

NTILES = 16
PAIRS_PER_TILE = P // NTILES
CHUNKS = PAIRS_PER_TILE // 16
ZROWS = NN // NTILES
ZBUF = 16384
NZDMA = ZROWS // ZBUF
SCAT_B = 128
SCAT_N = PAIRS_PER_TILE // SCAT_B


def _tc_vals_body(lg_ref, ws_ref, bs_ref, wo_ref, bo_ref, wcs_ref, wco_ref,
                  bc_ref, sval_ref, oval_ref):
    x = lg_ref[...]
    m = jnp.max(x, axis=1, keepdims=True)
    e = jnp.exp(x - m)
    p = e / jnp.sum(e, axis=1, keepdims=True)
    hs = jnp.maximum(
        jnp.dot(p, ws_ref[...], preferred_element_type=jnp.float32) + bs_ref[...], 0.0)
    ho = jnp.maximum(
        jnp.dot(p, wo_ref[...], preferred_element_type=jnp.float32) + bo_ref[...], 0.0)
    sval_ref[...] = jnp.sum(hs * wcs_ref[...], axis=1, keepdims=True) + bc_ref[0, 0]
    oval_ref[...] = jnp.sum(ho * wco_ref[...], axis=1, keepdims=True)


_tc_vals = pl.pallas_call(
    _tc_vals_body,
    out_shape=(jax.ShapeDtypeStruct((N_OBJ, 1), jnp.float32),
               jax.ShapeDtypeStruct((N_OBJ, 1), jnp.float32)),
)


def _sc_body(pairs_hbm, sval_hbm, oval_hbm, logits_hbm, mat_hbm,
             pair_v, stab_v, otab_v, logit_v, score_v, fidx_v, zero_v, dump_v,
             zsem, ssem):
    w = lax.axis_index("s")

    with jax.named_scope("zfill"):
        def zfill(k, carry):
            zero_v[pl.ds(k * 16, 16)] = jnp.zeros((16,), jnp.float32)
            return carry
        lax.fori_loop(0, ZBUF // 16, zfill, 0)

    zbase = w * ZROWS
    zcopies = [
        pltpu.async_copy(zero_v, mat_hbm.at[pl.ds(zbase + z * ZBUF, ZBUF)], zsem)
        for z in range(NZDMA)
    ]

    with jax.named_scope("stage"):
        pltpu.sync_copy(sval_hbm, stab_v)
        pltpu.sync_copy(oval_hbm, otab_v)
        pltpu.sync_copy(
            pairs_hbm.at[pl.ds(w * (PAIRS_PER_TILE * 2 // 128), PAIRS_PER_TILE * 2 // 128), :],
            pair_v)

    def body(m, carry):
        lane = lax.iota(jnp.int32, 16)
        flat = m * 32 + 2 * lane
        ii = plsc.load_gather(pair_v, [flat // 128, flat % 128])
        jj = plsc.load_gather(pair_v, [(flat + 1) // 128, (flat + 1) % 128])
        sv = plsc.load_gather(stab_v, [ii])
        ov = plsc.load_gather(otab_v, [jj])
        lg = sv + ov
        logit_v[pl.ds(m * 16, 16)] = lg
        sc = 1.0 / (1.0 + jnp.exp(-lg))
        r = m // 8
        o = (m % 8) * 16
        score_v[r, pl.ds(o, 16)] = sc
        fidx_v[r, pl.ds(o, 16)] = ii * N_OBJ + jj
        return carry
    with jax.named_scope("compute"):
        lax.fori_loop(0, CHUNKS, body, 0)

    with jax.named_scope("logits_out"):
        pltpu.sync_copy(
            logit_v, logits_hbm.at[pl.ds(w * PAIRS_PER_TILE, PAIRS_PER_TILE)])

    with jax.named_scope("zwait"):
        for d in zcopies:
            d.wait()
        plsc.subcore_barrier()

    with jax.named_scope("scatter"):
        def sbody(cc, carry):
            pltpu.async_copy(score_v.at[cc], mat_hbm.at[fidx_v.at[cc]], ssem)
            return carry
        lax.fori_loop(0, SCAT_N, sbody, 0)
        pltpu.make_async_copy(
            logits_hbm.at[pl.ds(0, SCAT_N * SCAT_B)], dump_v, ssem).wait()


_sc_scatter = functools.partial(
    pl.kernel,
    out_type=(jax.ShapeDtypeStruct((P,), jnp.float32),
              jax.ShapeDtypeStruct((NN,), jnp.float32)),
    mesh=plsc.VectorSubcoreMesh(
        core_axis_name="c", subcore_axis_name="s", num_cores=1),
    compiler_params=pltpu.CompilerParams(needs_layout_passes=False),
    scratch_types=(
        pltpu.VMEM((PAIRS_PER_TILE * 2 // 128, 128), jnp.int32),
        pltpu.VMEM((N_OBJ,), jnp.float32),
        pltpu.VMEM((N_OBJ,), jnp.float32),
        pltpu.VMEM((PAIRS_PER_TILE,), jnp.float32),
        pltpu.VMEM((SCAT_N, SCAT_B), jnp.float32),
        pltpu.VMEM((SCAT_N, SCAT_B), jnp.int32),
        pltpu.VMEM((ZBUF,), jnp.float32),
        pltpu.VMEM((SCAT_N * SCAT_B,), jnp.float32),
        pltpu.SemaphoreType.DMA,
        pltpu.SemaphoreType.DMA,
    ),
)(_sc_body)


def kernel(visual_feat, pred_logits, pair_idx, W_sub, b_sub, W_obj, b_obj,
           W_cls, b_cls):
    del visual_feat
    ws_cls = W_cls[:HIDDEN].reshape(1, HIDDEN)
    wo_cls = W_cls[HIDDEN:].reshape(1, HIDDEN)
    sval, oval = _tc_vals(pred_logits, W_sub, b_sub.reshape(1, HIDDEN),
                          W_obj, b_obj.reshape(1, HIDDEN),
                          ws_cls, wo_cls, b_cls.reshape(1, 1))
    logits, mat = _sc_scatter(pair_idx.reshape(P * 2 // 128, 128),
                              sval.reshape(-1), oval.reshape(-1))
    return logits, mat.reshape(N_OBJ, N_OBJ)


# speedup vs baseline: 1.0011x; 1.0011x over previous
import functools

import jax
import jax.numpy as jnp
from jax import lax
from jax.experimental import pallas as pl
from jax.experimental.pallas import tpu as pltpu
from jax.experimental.pallas import tpu_sc as plsc

N_OBJ = 2048
NUM_CLS = 151
HIDDEN = 256
P = 131072
NN = N_OBJ * N_OBJ

NTILES = 16
PAIRS_PER_TILE = P // NTILES
CHUNKS = PAIRS_PER_TILE // 16
ZROWS = NN // NTILES
ZBUF = 16384
NZDMA = ZROWS // ZBUF
SCAT_B = 128
SCAT_N = PAIRS_PER_TILE // SCAT_B


def _tc_vals_body(lg_ref, ws_ref, bs_ref, wo_ref, bo_ref, wcs_ref, wco_ref,
                  bc_ref, sval_ref, oval_ref):
    x = lg_ref[...]
    m = jnp.max(x, axis=1, keepdims=True)
    e = jnp.exp(x - m)
    p = e / jnp.sum(e, axis=1, keepdims=True)
    hs = jnp.maximum(
        jnp.dot(p, ws_ref[...], preferred_element_type=jnp.float32) + bs_ref[...], 0.0)
    ho = jnp.maximum(
        jnp.dot(p, wo_ref[...], preferred_element_type=jnp.float32) + bo_ref[...], 0.0)
    sval_ref[...] = jnp.sum(hs * wcs_ref[...], axis=1, keepdims=True) + bc_ref[0, 0]
    oval_ref[...] = jnp.sum(ho * wco_ref[...], axis=1, keepdims=True)


_tc_vals = pl.pallas_call(
    _tc_vals_body,
    out_shape=(jax.ShapeDtypeStruct((N_OBJ, 1), jnp.float32),
               jax.ShapeDtypeStruct((N_OBJ, 1), jnp.float32)),
)


def _sc_body(pairs_hbm, sval_hbm, oval_hbm, logits_hbm, mat_hbm,
             pair_v, stab_v, otab_v, logit_v, score_v, fidx_v, zero_v, dump_v,
             zsem, ssem):
    w = lax.axis_index("s")

    with jax.named_scope("zfill"):
        def zfill(k, carry):
            zero_v[pl.ds(k * 16, 16)] = jnp.zeros((16,), jnp.float32)
            return carry
        lax.fori_loop(0, ZBUF // 16, zfill, 0)

    zbase = w * ZROWS
    zcopies = [
        pltpu.async_copy(zero_v, mat_hbm.at[pl.ds(zbase + z * ZBUF, ZBUF)], zsem)
        for z in range(NZDMA)
    ]

    with jax.named_scope("stage"):
        pltpu.sync_copy(sval_hbm, stab_v)
        pltpu.sync_copy(oval_hbm, otab_v)
        pltpu.sync_copy(
            pairs_hbm.at[pl.ds(w * (PAIRS_PER_TILE * 2 // 128), PAIRS_PER_TILE * 2 // 128), :],
            pair_v)

    def body(m, carry):
        lane = lax.iota(jnp.int32, 16)
        flat = m * 32 + 2 * lane
        ii = plsc.load_gather(pair_v, [flat // 128, flat % 128])
        jj = plsc.load_gather(pair_v, [(flat + 1) // 128, (flat + 1) % 128])
        sv = plsc.load_gather(stab_v, [ii])
        ov = plsc.load_gather(otab_v, [jj])
        lg = sv + ov
        logit_v[pl.ds(m * 16, 16)] = lg
        sc = 1.0 / (1.0 + jnp.exp(-lg))
        score_v[pl.ds(m * 16, 16)] = sc
        fidx_v[pl.ds(m * 16, 16)] = ii * N_OBJ + jj
        return carry
    with jax.named_scope("compute"):
        lax.fori_loop(0, CHUNKS, body, 0)

    with jax.named_scope("logits_out"):
        pltpu.sync_copy(
            logit_v, logits_hbm.at[pl.ds(w * PAIRS_PER_TILE, PAIRS_PER_TILE)])

    with jax.named_scope("zwait"):
        for d in zcopies:
            d.wait()
        plsc.subcore_barrier()

    with jax.named_scope("scatter"):
        pltpu.async_copy(score_v, mat_hbm.at[fidx_v], ssem).wait()


_sc_scatter = functools.partial(
    pl.kernel,
    out_type=(jax.ShapeDtypeStruct((P,), jnp.float32),
              jax.ShapeDtypeStruct((NN,), jnp.float32)),
    mesh=plsc.VectorSubcoreMesh(
        core_axis_name="c", subcore_axis_name="s", num_cores=1),
    compiler_params=pltpu.CompilerParams(needs_layout_passes=False),
    scratch_types=(
        pltpu.VMEM((PAIRS_PER_TILE * 2 // 128, 128), jnp.int32),
        pltpu.VMEM((N_OBJ,), jnp.float32),
        pltpu.VMEM((N_OBJ,), jnp.float32),
        pltpu.VMEM((PAIRS_PER_TILE,), jnp.float32),
        pltpu.VMEM((PAIRS_PER_TILE,), jnp.float32),
        pltpu.VMEM((PAIRS_PER_TILE,), jnp.int32),
        pltpu.VMEM((ZBUF,), jnp.float32),
        pltpu.VMEM((PAIRS_PER_TILE,), jnp.float32),
        pltpu.SemaphoreType.DMA,
        pltpu.SemaphoreType.DMA,
    ),
)(_sc_body)


def kernel(visual_feat, pred_logits, pair_idx, W_sub, b_sub, W_obj, b_obj,
           W_cls, b_cls):
    del visual_feat
    ws_cls = W_cls[:HIDDEN].reshape(1, HIDDEN)
    wo_cls = W_cls[HIDDEN:].reshape(1, HIDDEN)
    sval, oval = _tc_vals(pred_logits, W_sub, b_sub.reshape(1, HIDDEN),
                          W_obj, b_obj.reshape(1, HIDDEN),
                          ws_cls, wo_cls, b_cls.reshape(1, 1))
    logits, mat = _sc_scatter(pair_idx.reshape(P * 2 // 128, 128),
                              sval.reshape(-1), oval.reshape(-1))
    return logits, mat.reshape(N_OBJ, N_OBJ)


# speedup vs baseline: 1.0237x; 1.0226x over previous
import functools

import jax
import jax.numpy as jnp
from jax import lax
from jax.experimental import pallas as pl
from jax.experimental.pallas import tpu as pltpu
from jax.experimental.pallas import tpu_sc as plsc

N_OBJ = 2048
NUM_CLS = 151
HIDDEN = 256
P = 131072
NN = N_OBJ * N_OBJ

NW = 32
PPW = P // NW
PROWS = PPW * 2 // 128
ZGRID = 32


def _tc_vals_body(lg_ref, ws_ref, bs_ref, wo_ref, bo_ref, wcs_ref, wco_ref,
                  bc_ref, sval_ref, oval_ref):
    x = lg_ref[...]
    m = jnp.max(x, axis=1, keepdims=True)
    e = jnp.exp(x - m)
    p = e / jnp.sum(e, axis=1, keepdims=True)
    hs = jnp.maximum(
        jnp.dot(p, ws_ref[...], preferred_element_type=jnp.float32) + bs_ref[...], 0.0)
    ho = jnp.maximum(
        jnp.dot(p, wo_ref[...], preferred_element_type=jnp.float32) + bo_ref[...], 0.0)
    sval_ref[...] = jnp.sum(hs * wcs_ref[...], axis=1, keepdims=True) + bc_ref[0, 0]
    oval_ref[...] = jnp.sum(ho * wco_ref[...], axis=1, keepdims=True)


_tc_vals = pl.pallas_call(
    _tc_vals_body,
    out_shape=(jax.ShapeDtypeStruct((N_OBJ, 1), jnp.float32),
               jax.ShapeDtypeStruct((N_OBJ, 1), jnp.float32)),
)


def _tc_zeros_body(z_ref):
    z_ref[...] = jnp.zeros_like(z_ref)


_tc_zeros = pl.pallas_call(
    _tc_zeros_body,
    grid=(ZGRID,),
    out_specs=pl.BlockSpec((NN // ZGRID,), lambda i: (i,)),
    out_shape=jax.ShapeDtypeStruct((NN,), jnp.float32),
)


def _sc_body(pairs_hbm, sval_hbm, oval_hbm, mat_hbm, logits_hbm,
             pair_v, stab_v, otab_v, logit_v, score_v, fidx_v, stsem, ssem):
    c = lax.axis_index("c")
    s = lax.axis_index("s")
    w = c * 16 + s

    with jax.named_scope("stage"):
        d1 = pltpu.async_copy(
            pairs_hbm.at[pl.ds(w * PROWS, PROWS), :], pair_v, stsem)
        d2 = pltpu.async_copy(sval_hbm, stab_v, stsem)
        d3 = pltpu.async_copy(oval_hbm, otab_v, stsem)
        d1.wait()
        d2.wait()
        d3.wait()

    def body(m, carry):
        lane = lax.iota(jnp.int32, 16)
        flat = m * 32 + 2 * lane
        ii = plsc.load_gather(pair_v, [flat // 128, flat % 128])
        jj = plsc.load_gather(pair_v, [(flat + 1) // 128, (flat + 1) % 128])
        sv = plsc.load_gather(stab_v, [ii])
        ov = plsc.load_gather(otab_v, [jj])
        lg = sv + ov
        logit_v[pl.ds(m * 16, 16)] = lg
        sc = 1.0 / (1.0 + jnp.exp(-lg))
        score_v[pl.ds(m * 16, 16)] = sc
        fidx_v[pl.ds(m * 16, 16)] = ii * N_OBJ + jj
        return carry
    with jax.named_scope("compute"):
        lax.fori_loop(0, PPW // 16, body, 0)

    with jax.named_scope("logits_out"):
        pltpu.sync_copy(logit_v, logits_hbm.at[pl.ds(w * PPW, PPW)])

    with jax.named_scope("scatter"):
        pltpu.async_copy(score_v, mat_hbm.at[fidx_v], ssem).wait()


_sc_scatter = functools.partial(
    pl.kernel,
    out_type=jax.ShapeDtypeStruct((P,), jnp.float32),
    mesh=plsc.VectorSubcoreMesh(core_axis_name="c", subcore_axis_name="s"),
    compiler_params=pltpu.CompilerParams(needs_layout_passes=False),
    scratch_types=(
        pltpu.VMEM((PROWS, 128), jnp.int32),
        pltpu.VMEM((N_OBJ,), jnp.float32),
        pltpu.VMEM((N_OBJ,), jnp.float32),
        pltpu.VMEM((PPW,), jnp.float32),
        pltpu.VMEM((PPW,), jnp.float32),
        pltpu.VMEM((PPW,), jnp.int32),
        pltpu.SemaphoreType.DMA,
        pltpu.SemaphoreType.DMA,
    ),
)(_sc_body)


def kernel(visual_feat, pred_logits, pair_idx, W_sub, b_sub, W_obj, b_obj,
           W_cls, b_cls):
    del visual_feat
    ws_cls = W_cls[:HIDDEN].reshape(1, HIDDEN)
    wo_cls = W_cls[HIDDEN:].reshape(1, HIDDEN)
    sval, oval = _tc_vals(pred_logits, W_sub, b_sub.reshape(1, HIDDEN),
                          W_obj, b_obj.reshape(1, HIDDEN),
                          ws_cls, wo_cls, b_cls.reshape(1, 1))
    zmat = _tc_zeros()
    mat_ref = jax.new_ref(zmat)
    logits = _sc_scatter(pair_idx.reshape(P * 2 // 128, 128),
                         sval.reshape(-1), oval.reshape(-1), mat_ref)
    return logits, mat_ref[...].reshape(N_OBJ, N_OBJ)
